# baseline (speedup 1.0000x reference)
import functools

import jax
import jax.numpy as jnp
from jax import lax
from jax.experimental import pallas as pl
from jax.experimental.pallas import tpu as pltpu
from jax.experimental.pallas import tpu_sc as plsc

N = 10000
D = 128
E = 320000
C = 125
NW = 32
CH_PER_W = E // (C * NW)
BLK = 16
N_BLK = CH_PER_W // BLK
SEG = 80
NSEG_TOT = N // SEG

_MESH = plsc.VectorSubcoreMesh(core_axis_name="c", subcore_axis_name="s")


@functools.partial(
    pl.kernel,
    out_type=jax.ShapeDtypeStruct((2, N, D), jnp.float32),
    mesh=_MESH,
    scratch_types=[
        pltpu.VMEM((BLK, C), jnp.int32),
        pltpu.VMEM((BLK, C), jnp.int32),
        pltpu.VMEM((BLK, C), jnp.int32),
        pltpu.VMEM((BLK, C), jnp.int32),
        pltpu.VMEM((C, D), jnp.float32),
        pltpu.VMEM((C, D), jnp.float32),
        pltpu.VMEM_SHARED((N, D), jnp.float32),
        pltpu.SemaphoreType.DMA,
        pltpu.SemaphoreType.DMA,
        pltpu.SemaphoreType.DMA,
        pltpu.SemaphoreType.DMA,
        pltpu.SemaphoreType.DMA,
    ],
)
def _sc_agg(table, src2d, dst2d, out,
            sidxA, didxA, sidxB, didxB, r0, r1, acc, sem0, sem1, semi,
            semz0, semz1):
    c = lax.axis_index("c")
    s = lax.axis_index("s")
    wid = c * 16 + s
    base = wid * CH_PER_W

    zv = jnp.zeros((16,), jnp.float32)

    def zrow(i, carry):
        for k in range(D // 16):
            r0[i, pl.ds(k * 16, 16)] = zv
        return carry

    lax.fori_loop(0, SEG, zrow, 0)
    for g in range(8):
        seg = s + 16 * g

        @pl.when(seg < NSEG_TOT)
        def _():
            pltpu.async_copy(r0.at[pl.ds(0, SEG)],
                             acc.at[pl.ds(seg * SEG, SEG)], semz0)

    for g in range(8):
        seg = s + 16 * g

        @pl.when(seg < NSEG_TOT)
        def _():
            pltpu.make_async_copy(r0.at[pl.ds(0, SEG)],
                                  acc.at[pl.ds(seg * SEG, SEG)], semz0).wait()

    pltpu.sync_copy(src2d.at[pl.ds(base, BLK)], sidxA)
    pltpu.sync_copy(dst2d.at[pl.ds(base, BLK)], didxA)
    pltpu.async_copy(table.at[sidxA.at[0]], r0, sem0)
    pltpu.async_copy(table.at[sidxA.at[1]], r1, sem1)

    plsc.subcore_barrier()

    idx_bufs = [(sidxA, didxA), (sidxB, didxB)]
    for b in range(N_BLK):
        sidx, didx = idx_bufs[b % 2]
        nsidx, ndidx = idx_bufs[(b + 1) % 2]
        if b + 1 < N_BLK:
            nbase = base + (b + 1) * BLK
            pltpu.async_copy(src2d.at[pl.ds(nbase, BLK)], nsidx, semi)
            pltpu.async_copy(dst2d.at[pl.ds(nbase, BLK)], ndidx, semi)

        def pair(i, carry, sidx=sidx, didx=didx):
            j = 2 * i
            pltpu.make_async_copy(table.at[sidx.at[j]], r0, sem0).wait()
            pltpu.async_copy(r0, acc.at[didx.at[j]], semz0, add=True)
            pltpu.make_async_copy(table.at[sidx.at[j + 1]], r1, sem1).wait()
            pltpu.async_copy(r1, acc.at[didx.at[j + 1]], semz1, add=True)
            pltpu.make_async_copy(r0, acc.at[didx.at[j]], semz0).wait()
            pltpu.async_copy(table.at[sidx.at[j + 2]], r0, sem0)
            pltpu.make_async_copy(r1, acc.at[didx.at[j + 1]], semz1).wait()
            pltpu.async_copy(table.at[sidx.at[j + 3]], r1, sem1)
            return carry

        lax.fori_loop(0, BLK // 2 - 2, pair, 0)

        j = BLK - 4
        pltpu.make_async_copy(table.at[sidx.at[j]], r0, sem0).wait()
        pltpu.async_copy(r0, acc.at[didx.at[j]], semz0, add=True)
        pltpu.make_async_copy(table.at[sidx.at[j + 1]], r1, sem1).wait()
        pltpu.async_copy(r1, acc.at[didx.at[j + 1]], semz1, add=True)
        pltpu.make_async_copy(r0, acc.at[didx.at[j]], semz0).wait()
        pltpu.async_copy(table.at[sidx.at[j + 2]], r0, sem0)
        pltpu.make_async_copy(r1, acc.at[didx.at[j + 1]], semz1).wait()
        pltpu.async_copy(table.at[sidx.at[j + 3]], r1, sem1)

        j = BLK - 2
        pltpu.make_async_copy(table.at[sidx.at[j]], r0, sem0).wait()
        pltpu.async_copy(r0, acc.at[didx.at[j]], semz0, add=True)
        pltpu.make_async_copy(table.at[sidx.at[j + 1]], r1, sem1).wait()
        pltpu.async_copy(r1, acc.at[didx.at[j + 1]], semz1, add=True)
        if b + 1 < N_BLK:
            nbase = base + (b + 1) * BLK
            pltpu.make_async_copy(src2d.at[pl.ds(nbase, BLK)], nsidx, semi).wait()
            pltpu.make_async_copy(dst2d.at[pl.ds(nbase, BLK)], ndidx, semi).wait()
            pltpu.make_async_copy(r0, acc.at[didx.at[j]], semz0).wait()
            pltpu.async_copy(table.at[nsidx.at[0]], r0, sem0)
            pltpu.make_async_copy(r1, acc.at[didx.at[j + 1]], semz1).wait()
            pltpu.async_copy(table.at[nsidx.at[1]], r1, sem1)
        else:
            pltpu.make_async_copy(r0, acc.at[didx.at[j]], semz0).wait()
            pltpu.make_async_copy(r1, acc.at[didx.at[j + 1]], semz1).wait()

    plsc.subcore_barrier()

    dbuf = [(r0, semz0), (r1, semz1)]
    for g in range(8):
        seg = s + 16 * g
        rb, semd = dbuf[g % 2]

        if g >= 2:
            pseg = s + 16 * (g - 2)

            @pl.when(pseg < NSEG_TOT)
            def _(rb=rb, semd=semd, pseg=pseg):
                pltpu.make_async_copy(rb.at[pl.ds(0, SEG)],
                                      out.at[c, pl.ds(pseg * SEG, SEG)],
                                      semd).wait()

        @pl.when(seg < NSEG_TOT)
        def _(rb=rb, semd=semd, seg=seg):
            pltpu.sync_copy(acc.at[pl.ds(seg * SEG, SEG)], rb.at[pl.ds(0, SEG)])
            pltpu.async_copy(rb.at[pl.ds(0, SEG)],
                             out.at[c, pl.ds(seg * SEG, SEG)], semd)

    for g in range(6, 8):
        seg = s + 16 * g
        rb, semd = dbuf[g % 2]

        @pl.when(seg < NSEG_TOT)
        def _(rb=rb, semd=semd, seg=seg):
            pltpu.make_async_copy(rb.at[pl.ds(0, SEG)],
                                  out.at[c, pl.ds(seg * SEG, SEG)], semd).wait()


def _prelu(x, a):
    return jnp.where(x >= 0.0, x, a * x)


_RB = 1000
_GRID = N // _RB


def _l1_body(p0, p1, lp0, lp1, ln0, ln1, xp, xl,
             wt_lip, wb_lip, b_lip, a_lip,
             wt_pul, wb_pul, b_pul, a_pul,
             wt_nhl, wb_nhl, b_nhl, a_nhl,
             h_path, h_link):
    f32 = jnp.float32
    hp = jnp.dot((p0[...] + p1[...]), wt_lip[...], preferred_element_type=f32)
    hp = hp + jnp.dot(xp[...], wb_lip[...], preferred_element_type=f32)
    h_path[...] = _prelu(hp + b_lip[...], a_lip[...])

    t1 = jnp.dot((lp0[...] + lp1[...]), wt_pul[...], preferred_element_type=f32)
    t1 = t1 + jnp.dot(xl[...], wb_pul[...], preferred_element_type=f32)
    t1 = _prelu(t1 + b_pul[...], a_pul[...])
    t2 = jnp.dot((ln0[...] + ln1[...]), wt_nhl[...], preferred_element_type=f32)
    t2 = t2 + jnp.dot(xl[...], wb_nhl[...], preferred_element_type=f32)
    t2 = _prelu(t2 + b_nhl[...], a_nhl[...])
    h_link[...] = t1 + t2


def _l2_body(q0, q1, hp, w2, b2, a2, k2,
             w0, b0, w1, b1, wf, bf, out):
    f32 = jnp.float32
    t = q0[...] + q1[...] + k2[...] * hp[...]
    g = _prelu(jnp.dot(t, w2[...], preferred_element_type=f32) + b2[...], a2[...])
    y = jnp.maximum(jnp.dot(g, w0[...], preferred_element_type=f32) + b0[...], 0.0)
    y = jnp.maximum(jnp.dot(y, w1[...], preferred_element_type=f32) + b1[...], 0.0)
    out[...] = jnp.dot(y, wf[...], preferred_element_type=f32) + bf[...]


def _row_spec():
    return pl.BlockSpec((_RB, D), lambda i: (i, 0))


def _w_spec(shape):
    return pl.BlockSpec(shape, lambda i: tuple(0 for _ in shape))


def kernel(x_path, x_link, x_node, ei_pul, ei_lip, ei_lcn, ei_nhl, path_batch, params):
    del ei_lcn, path_batch
    f32 = jnp.float32

    src_lip = ei_lip[0].reshape(E // C, C)
    dst_lip = ei_lip[1].reshape(E // C, C)
    src_pul = ei_pul[0].reshape(E // C, C)
    dst_pul = ei_pul[1].reshape(E // C, C)
    src_nhl = ei_nhl[0].reshape(E // C, C)
    dst_nhl = ei_nhl[1].reshape(E // C, C)

    aggP = _sc_agg(x_link, src_lip, dst_lip)
    aggLp = _sc_agg(x_path, src_pul, dst_pul)
    aggLn = _sc_agg(x_node, src_nhl, dst_nhl)

    def gin_w(p):
        wt = p["W"][:D]
        wb = (1.0 + p["eps"][0]) * p["W"][D:]
        b = p["b"].reshape(1, D)
        a = jnp.full((1, D), p["a"][0], f32)
        return wt, wb, b, a

    c1 = params["c1"]
    wt_lip, wb_lip, b_lip, a_lip = gin_w(c1["lip"])
    wt_pul, wb_pul, b_pul, a_pul = gin_w(c1["pul"])
    wt_nhl, wb_nhl, b_nhl, a_nhl = gin_w(c1["nhl"])

    wspec128 = _w_spec((D, D))
    bspec = _w_spec((1, D))
    row = _row_spec()

    h_path, h_link = pl.pallas_call(
        _l1_body,
        grid=(_GRID,),
        in_specs=[row] * 8 + [wspec128, wspec128, bspec, bspec] * 3,
        out_specs=[row, row],
        out_shape=[jax.ShapeDtypeStruct((N, D), f32)] * 2,
    )(aggP[0], aggP[1], aggLp[0], aggLp[1], aggLn[0], aggLn[1], x_path, x_link,
      wt_lip, wb_lip, b_lip, a_lip,
      wt_pul, wb_pul, b_pul, a_pul,
      wt_nhl, wb_nhl, b_nhl, a_nhl)

    aggP2 = _sc_agg(h_link, src_lip, dst_lip)

    c2 = params["c2"]["lip"]
    w2 = c2["W"]
    b2 = c2["b"].reshape(1, D)
    a2 = jnp.full((1, D), c2["a"][0], f32)
    k2 = jnp.full((1, D), 1.0 + c2["eps"][0], f32)
    mlp = params["mlp"]
    w0, b0 = mlp[0]["W"], mlp[0]["b"].reshape(1, 256)
    w1, b1 = mlp[1]["W"], mlp[1]["b"].reshape(1, D)
    wf, bf = mlp[2]["W"], mlp[2]["b"].reshape(1, 1)

    out = pl.pallas_call(
        _l2_body,
        grid=(_GRID,),
        in_specs=[row, row, row,
                  wspec128, bspec, bspec, bspec,
                  _w_spec((D, 256)), _w_spec((1, 256)),
                  _w_spec((256, D)), bspec,
                  _w_spec((D, 1)), _w_spec((1, 1))],
        out_specs=pl.BlockSpec((_RB, 1), lambda i: (i, 0)),
        out_shape=jax.ShapeDtypeStruct((N, 1), f32),
    )(aggP2[0], aggP2[1], h_path, w2, b2, a2, k2, w0, b0, w1, b1, wf, bf)

    return out


# speedup vs baseline: 1.1070x; 1.1070x over previous
import functools

import jax
import jax.numpy as jnp
from jax import lax
from jax.experimental import pallas as pl
from jax.experimental.pallas import tpu as pltpu
from jax.experimental.pallas import tpu_sc as plsc

N = 10000
D = 128
E = 320000
C = 125
NW = 32
CH_PER_W = E // (C * NW)
BLK = 16
N_BLK = CH_PER_W // BLK
SEG = 80
NSEG_TOT = N // SEG

_MESH = plsc.VectorSubcoreMesh(core_axis_name="c", subcore_axis_name="s")


@functools.partial(
    pl.kernel,
    out_type=jax.ShapeDtypeStruct((2, N, D), jnp.float32),
    mesh=_MESH,
    scratch_types=[
        pltpu.VMEM((BLK, C), jnp.int32),
        pltpu.VMEM((BLK, C), jnp.int32),
        pltpu.VMEM((BLK, C), jnp.int32),
        pltpu.VMEM((BLK, C), jnp.int32),
        pltpu.VMEM((C, D), jnp.float32),
        pltpu.VMEM((C, D), jnp.float32),
        pltpu.VMEM_SHARED((N, D), jnp.float32),
        pltpu.SemaphoreType.DMA,
        pltpu.SemaphoreType.DMA,
        pltpu.SemaphoreType.DMA,
        pltpu.SemaphoreType.DMA,
        pltpu.SemaphoreType.DMA,
    ],
)
def _sc_agg(table, src2d, dst2d, out,
            sidxA, didxA, sidxB, didxB, r0, r1, acc, sem0, sem1, semi,
            semz0, semz1):
    c = lax.axis_index("c")
    s = lax.axis_index("s")
    wid = c * 16 + s
    base = wid * CH_PER_W

    zv = jnp.zeros((16,), jnp.float32)

    def zrow(i, carry):
        for k in range(D // 16):
            r0[i, pl.ds(k * 16, 16)] = zv
        return carry

    lax.fori_loop(0, SEG, zrow, 0)
    for g in range(8):
        seg = s + 16 * g

        @pl.when(seg < NSEG_TOT)
        def _():
            pltpu.async_copy(r0.at[pl.ds(0, SEG)],
                             acc.at[pl.ds(seg * SEG, SEG)], semz0)

    for g in range(8):
        seg = s + 16 * g

        @pl.when(seg < NSEG_TOT)
        def _():
            pltpu.make_async_copy(r0.at[pl.ds(0, SEG)],
                                  acc.at[pl.ds(seg * SEG, SEG)], semz0).wait()

    pltpu.sync_copy(src2d.at[pl.ds(base, BLK)], sidxA)
    pltpu.sync_copy(dst2d.at[pl.ds(base, BLK)], didxA)
    pltpu.async_copy(table.at[sidxA.at[0]], r0, sem0)

    plsc.subcore_barrier()

    idx_bufs = [(sidxA, didxA), (sidxB, didxB)]
    for b in range(N_BLK):
        sidx, didx = idx_bufs[b % 2]
        nsidx, ndidx = idx_bufs[(b + 1) % 2]
        if b + 1 < N_BLK:
            nbase = base + (b + 1) * BLK
            pltpu.async_copy(src2d.at[pl.ds(nbase, BLK)], nsidx, semi)
            pltpu.async_copy(dst2d.at[pl.ds(nbase, BLK)], ndidx, semi)

        def pair(i, carry, sidx=sidx, didx=didx):
            j = 2 * i
            pltpu.make_async_copy(table.at[sidx.at[j]], r0, sem0).wait()
            pltpu.async_copy(table.at[sidx.at[j + 1]], r1, sem1)
            pltpu.sync_copy(r0, acc.at[didx.at[j]], add=True)
            pltpu.make_async_copy(table.at[sidx.at[j + 1]], r1, sem1).wait()
            pltpu.async_copy(table.at[sidx.at[j + 2]], r0, sem0)
            pltpu.sync_copy(r1, acc.at[didx.at[j + 1]], add=True)
            return carry

        lax.fori_loop(0, BLK // 2 - 1, pair, 0)

        pltpu.make_async_copy(table.at[sidx.at[BLK - 2]], r0, sem0).wait()
        pltpu.async_copy(table.at[sidx.at[BLK - 1]], r1, sem1)
        pltpu.sync_copy(r0, acc.at[didx.at[BLK - 2]], add=True)
        if b + 1 < N_BLK:
            nbase = base + (b + 1) * BLK
            pltpu.make_async_copy(src2d.at[pl.ds(nbase, BLK)], nsidx, semi).wait()
            pltpu.make_async_copy(dst2d.at[pl.ds(nbase, BLK)], ndidx, semi).wait()
            pltpu.async_copy(table.at[nsidx.at[0]], r0, sem0)
        pltpu.make_async_copy(table.at[sidx.at[BLK - 1]], r1, sem1).wait()
        pltpu.sync_copy(r1, acc.at[didx.at[BLK - 1]], add=True)

    plsc.subcore_barrier()

    dbuf = [(r0, semz0), (r1, semz1)]
    for g in range(8):
        seg = s + 16 * g
        rb, semd = dbuf[g % 2]

        if g >= 2:
            pseg = s + 16 * (g - 2)

            @pl.when(pseg < NSEG_TOT)
            def _(rb=rb, semd=semd, pseg=pseg):
                pltpu.make_async_copy(rb.at[pl.ds(0, SEG)],
                                      out.at[c, pl.ds(pseg * SEG, SEG)],
                                      semd).wait()

        @pl.when(seg < NSEG_TOT)
        def _(rb=rb, semd=semd, seg=seg):
            pltpu.sync_copy(acc.at[pl.ds(seg * SEG, SEG)], rb.at[pl.ds(0, SEG)])
            pltpu.async_copy(rb.at[pl.ds(0, SEG)],
                             out.at[c, pl.ds(seg * SEG, SEG)], semd)

    for g in range(6, 8):
        seg = s + 16 * g
        rb, semd = dbuf[g % 2]

        @pl.when(seg < NSEG_TOT)
        def _(rb=rb, semd=semd, seg=seg):
            pltpu.make_async_copy(rb.at[pl.ds(0, SEG)],
                                  out.at[c, pl.ds(seg * SEG, SEG)], semd).wait()


def _prelu(x, a):
    return jnp.where(x >= 0.0, x, a * x)


_RB = 1000
_GRID = N // _RB


def _l1_body(p0, p1, lp0, lp1, ln0, ln1, xp, xl,
             wt_lip, wb_lip, b_lip, a_lip,
             wt_pul, wb_pul, b_pul, a_pul,
             wt_nhl, wb_nhl, b_nhl, a_nhl,
             h_path, h_link):
    f32 = jnp.float32
    hp = jnp.dot((p0[...] + p1[...]), wt_lip[...], preferred_element_type=f32)
    hp = hp + jnp.dot(xp[...], wb_lip[...], preferred_element_type=f32)
    h_path[...] = _prelu(hp + b_lip[...], a_lip[...])

    t1 = jnp.dot((lp0[...] + lp1[...]), wt_pul[...], preferred_element_type=f32)
    t1 = t1 + jnp.dot(xl[...], wb_pul[...], preferred_element_type=f32)
    t1 = _prelu(t1 + b_pul[...], a_pul[...])
    t2 = jnp.dot((ln0[...] + ln1[...]), wt_nhl[...], preferred_element_type=f32)
    t2 = t2 + jnp.dot(xl[...], wb_nhl[...], preferred_element_type=f32)
    t2 = _prelu(t2 + b_nhl[...], a_nhl[...])
    h_link[...] = t1 + t2


def _l2_body(q0, q1, hp, w2, b2, a2, k2,
             w0, b0, w1, b1, wf, bf, out):
    f32 = jnp.float32
    t = q0[...] + q1[...] + k2[...] * hp[...]
    g = _prelu(jnp.dot(t, w2[...], preferred_element_type=f32) + b2[...], a2[...])
    y = jnp.maximum(jnp.dot(g, w0[...], preferred_element_type=f32) + b0[...], 0.0)
    y = jnp.maximum(jnp.dot(y, w1[...], preferred_element_type=f32) + b1[...], 0.0)
    out[...] = jnp.dot(y, wf[...], preferred_element_type=f32) + bf[...]


def _row_spec():
    return pl.BlockSpec((_RB, D), lambda i: (i, 0))


def _w_spec(shape):
    return pl.BlockSpec(shape, lambda i: tuple(0 for _ in shape))


def kernel(x_path, x_link, x_node, ei_pul, ei_lip, ei_lcn, ei_nhl, path_batch, params):
    del ei_lcn, path_batch
    f32 = jnp.float32

    src_lip = ei_lip[0].reshape(E // C, C)
    dst_lip = ei_lip[1].reshape(E // C, C)
    src_pul = ei_pul[0].reshape(E // C, C)
    dst_pul = ei_pul[1].reshape(E // C, C)
    src_nhl = ei_nhl[0].reshape(E // C, C)
    dst_nhl = ei_nhl[1].reshape(E // C, C)

    aggP = _sc_agg(x_link, src_lip, dst_lip)
    aggLp = _sc_agg(x_path, src_pul, dst_pul)
    aggLn = _sc_agg(x_node, src_nhl, dst_nhl)

    def gin_w(p):
        wt = p["W"][:D]
        wb = (1.0 + p["eps"][0]) * p["W"][D:]
        b = p["b"].reshape(1, D)
        a = jnp.full((1, D), p["a"][0], f32)
        return wt, wb, b, a

    c1 = params["c1"]
    wt_lip, wb_lip, b_lip, a_lip = gin_w(c1["lip"])
    wt_pul, wb_pul, b_pul, a_pul = gin_w(c1["pul"])
    wt_nhl, wb_nhl, b_nhl, a_nhl = gin_w(c1["nhl"])

    wspec128 = _w_spec((D, D))
    bspec = _w_spec((1, D))
    row = _row_spec()

    h_path, h_link = pl.pallas_call(
        _l1_body,
        grid=(_GRID,),
        in_specs=[row] * 8 + [wspec128, wspec128, bspec, bspec] * 3,
        out_specs=[row, row],
        out_shape=[jax.ShapeDtypeStruct((N, D), f32)] * 2,
    )(aggP[0], aggP[1], aggLp[0], aggLp[1], aggLn[0], aggLn[1], x_path, x_link,
      wt_lip, wb_lip, b_lip, a_lip,
      wt_pul, wb_pul, b_pul, a_pul,
      wt_nhl, wb_nhl, b_nhl, a_nhl)

    aggP2 = _sc_agg(h_link, src_lip, dst_lip)

    c2 = params["c2"]["lip"]
    w2 = c2["W"]
    b2 = c2["b"].reshape(1, D)
    a2 = jnp.full((1, D), c2["a"][0], f32)
    k2 = jnp.full((1, D), 1.0 + c2["eps"][0], f32)
    mlp = params["mlp"]
    w0, b0 = mlp[0]["W"], mlp[0]["b"].reshape(1, 256)
    w1, b1 = mlp[1]["W"], mlp[1]["b"].reshape(1, D)
    wf, bf = mlp[2]["W"], mlp[2]["b"].reshape(1, 1)

    out = pl.pallas_call(
        _l2_body,
        grid=(_GRID,),
        in_specs=[row, row, row,
                  wspec128, bspec, bspec, bspec,
                  _w_spec((D, 256)), _w_spec((1, 256)),
                  _w_spec((256, D)), bspec,
                  _w_spec((D, 1)), _w_spec((1, 1))],
        out_specs=pl.BlockSpec((_RB, 1), lambda i: (i, 0)),
        out_shape=jax.ShapeDtypeStruct((N, 1), f32),
    )(aggP2[0], aggP2[1], h_path, w2, b2, a2, k2, w0, b0, w1, b1, wf, bf)

    return out
